# baseline (speedup 1.0000x reference)
import functools

import jax
import jax.numpy as jnp
from jax import lax
from jax.experimental import pallas as pl
from jax.experimental.pallas import tpu as pltpu
from jax.experimental.pallas import tpu_sc as plsc

V = 1000001
D = 64
NB = 16384
NT = 50
NC, NS = 2, 16
NW = NC * NS
BPW = NB // NW
NBB = BPW // 128
HB = 2
CH = HB * 128
NCHUNK = NT * (NBB // HB)
PERIOD = 6
NFULL = 96


def _make_kernel():
    mesh = plsc.VectorSubcoreMesh(core_axis_name="c", subcore_axis_name="s")

    @functools.partial(
        pl.kernel,
        mesh=mesh,
        compiler_params=pltpu.CompilerParams(
            needs_layout_passes=False, use_tc_tiling_on_sc=False
        ),
        out_type=jax.ShapeDtypeStruct((NT, D // 8, NB // 128, 8, 128), jnp.float32),
        scratch_types=[
            pltpu.VMEM((HB, 128), jnp.int32),
            pltpu.VMEM((HB, 128), jnp.int32),
            pltpu.VMEM((HB, 128), jnp.int32),
            pltpu.VMEM((CH, D), jnp.float32),
            pltpu.VMEM((CH, D), jnp.float32),
            pltpu.VMEM((CH, D), jnp.float32),
            pltpu.VMEM((D // 8, HB, 8, 128), jnp.float32),
            pltpu.VMEM((D // 8, HB, 8, 128), jnp.float32),
            pltpu.SemaphoreType.DMA,
            pltpu.SemaphoreType.DMA,
            pltpu.SemaphoreType.DMA,
        ],
    )
    def gather_kernel(
        table_hbm, idx_hbm, out_hbm,
        idx0, idx1, idx2, rowsa, rowsb, rowsc, cbuf0, cbuf1,
        isem, gsem, osem,
    ):
        wid = lax.axis_index("s") * NC + lax.axis_index("c")
        bb0 = wid * NBB
        idxb = (idx0, idx1, idx2)
        rows = (rowsa, rowsb, rowsc)
        cbuf = (cbuf0, cbuf1)
        lane = lax.iota(jnp.int32, 16)

        idx_row_base = wid * NT * NBB

        def fire_idx(c, ip):
            t = c // 2
            h = c % 2
            pltpu.async_copy(
                idx_hbm.at[pl.ds(idx_row_base + t * NBB + h * HB, HB)],
                idxb[ip],
                isem,
            )

        def drain_idx():
            pltpu.make_async_copy(idx_hbm.at[pl.ds(0, HB)], idxb[0], isem).wait()

        def fire_gather(c, ip, rp):
            for k in range(HB):
                pltpu.async_copy(
                    table_hbm.at[idxb[ip].at[k]],
                    rows[rp].at[pl.ds(k * 128, 128)],
                    gsem,
                )

        def drain_gather(rp):
            pltpu.make_async_copy(
                table_hbm.at[pl.ds(0, CH)], rows[rp], gsem
            ).wait()

        def fire_out(c, cp):
            t = c // 2
            h = c % 2
            for cb in range(D // 8):
                pltpu.async_copy(
                    cbuf[cp].at[cb],
                    out_hbm.at[t, cb, pl.ds(bb0 + h * HB, HB)],
                    osem,
                )

        def drain_out(cp):
            for cb in range(D // 8):
                pltpu.make_async_copy(
                    cbuf[cp].at[cb], out_hbm.at[0, cb, pl.ds(0, HB)], osem
                ).wait()

        kcol = [k * 16 + lane for k in range(D // 16)]
        pcb = [(k * 16 + lane) // 8 for k in range(D // 16)]
        pci = [(k * 16 + lane) % 8 for k in range(D // 16)]

        def transpose_chunk(ip, rp, cp):
            @plsc.parallel_loop(0, HB * 8 * 4, unroll=2)
            def grp(i):
                g = i // 4
                k = i % 4
                bbl = g // 8
                bmg = g % 8
                bm_v = bmg * 16 + lane
                row_v = bbl * 128 + bm_v
                bbl_v = jnp.full((16,), bbl, jnp.int32)
                iv = plsc.load_gather(idxb[ip], [bbl_v, bm_v])
                m = jnp.where(iv == 0, jnp.float32(0.0), jnp.float32(1.0))
                c16 = k * 16
                for j in range(16):
                    col = c16 + j
                    x = plsc.load_gather(
                        rows[rp], [row_v, jnp.full((16,), col, jnp.int32)]
                    )
                    plsc.store_scatter(
                        cbuf[cp],
                        [
                            jnp.full((16,), col // 8, jnp.int32),
                            bbl_v,
                            jnp.full((16,), col % 8, jnp.int32),
                            bm_v,
                        ],
                        x * m,
                    )

        def step(c, j, *, tail=False):
            ip = j % 3
            rp = j % 3
            cp = j % 2
            ip2 = (j + 2) % 3
            rp2 = (j + 2) % 3
            if tail:
                if c + 2 < NCHUNK:
                    drain_idx()
                    fire_gather(c + 2, ip2, rp2)
                drain_out(cp)
            else:
                drain_idx()
                fire_gather(c + 2, ip2, rp2)
                pl.when(c >= 2)(lambda: drain_out(cp))
            drain_gather(rp)
            transpose_chunk(ip, rp, cp)
            fire_out(c, cp)
            if tail:
                if c + 3 < NCHUNK:
                    fire_idx(c + 3, j % 3)
            else:
                fire_idx(c + 3, j % 3)

        fire_idx(0, 0)
        fire_idx(1, 1)
        fire_idx(2, 2)
        drain_idx()
        fire_gather(0, 0, 0)
        drain_idx()
        fire_gather(1, 1, 1)

        def outer(u, carry):
            c0 = u * PERIOD
            for j in range(PERIOD):
                step(c0 + j, j)
            return carry

        lax.fori_loop(0, NFULL // PERIOD, outer, 0)
        for c in range(NFULL, NCHUNK):
            step(c, c % PERIOD, tail=True)
        drain_out((NCHUNK - 2) % 2)
        drain_out((NCHUNK - 1) % 2)

    return gather_kernel


@functools.lru_cache(maxsize=1)
def _get_kernel():
    return _make_kernel()


def kernel(ivectors, data):
    idx = (
        data.astype(jnp.int32)
        .T.reshape(NT, NW, NBB, 128)
        .transpose(1, 0, 2, 3)
        .reshape(NW * NT * NBB, 128)
    )
    out5 = _get_kernel()(ivectors, idx)
    return out5.transpose(2, 4, 0, 1, 3).reshape(NB, NT, D)
